# baseline (speedup 1.0000x reference)
import jax
import jax.numpy as jnp
from jax import lax
from jax.experimental import pallas as pl
from jax.experimental.pallas import tpu as pltpu
from jax.experimental.pallas import tpu_sc as plsc

BATCH = 16384
DIM = 64
N_CLASSES = 1000000
NUM_CORES = 2
NUM_SUBCORES = 16
NUM_WORKERS = NUM_CORES * NUM_SUBCORES
COLS_PER_WORKER = BATCH // NUM_WORKERS
LANES = 16
LANE_TILE = 128
RING = 11


def _sc_body(xt_hbm, cls_hbm, pt_hbm, out_hbm, cls_v, acc, bufs, colbuf, sems, csem):
    sid = lax.axis_index("s")
    wid = sid * NUM_CORES + lax.axis_index("c")
    base = pl.multiple_of(wid * COLS_PER_WORKER, COLS_PER_WORKER)
    pltpu.sync_copy(cls_hbm.at[wid], cls_v.at[:, pl.ds(0, COLS_PER_WORKER)])
    pltpu.sync_copy(xt_hbm.at[:, pl.ds(base, COLS_PER_WORKER)], acc)

    def fire(c, k):
        z = cls_v[0, pl.ds(c, LANES)][0]
        z128 = pl.multiple_of(
            lax.shift_left(lax.shift_right_logical(z, 7), 7), LANE_TILE
        )
        pltpu.async_copy(
            pt_hbm.at[:, pl.ds(z128, LANE_TILE)], bufs.at[sid, k], sems.at[k]
        )

    def drain(k):
        pltpu.make_async_copy(
            pt_hbm.at[:, pl.ds(0, LANE_TILE)], bufs.at[sid, k], sems.at[k]
        ).wait()

    for k in range(RING):
        fire(k, k)

    def col_body(c, carry):
        k = lax.rem(c, RING)
        drain(k)
        l = cls_v[0, pl.ds(c, LANES)][0] & (LANE_TILE - 1)
        pltpu.async_copy(bufs.at[sid, k, :, l], colbuf, csem).wait()
        cvec = jnp.full((LANES,), c, jnp.int32)
        for j in range(DIM // LANES):
            dvec = lax.iota(jnp.int32, LANES) + j * LANES
            v = colbuf[pl.ds(j * LANES, LANES)]
            plsc.addupdate_scatter(acc, [dvec, cvec], v)

        @pl.when(c + RING < COLS_PER_WORKER)
        def _():
            fire(c + RING, k)

        return carry

    lax.fori_loop(0, COLS_PER_WORKER, col_body, 0)
    pltpu.sync_copy(acc, out_hbm.at[:, pl.ds(base, COLS_PER_WORKER)])


@jax.jit
def kernel(x, classes, biases):
    cls_r = classes.astype(jnp.int32).reshape(NUM_WORKERS, 1, COLS_PER_WORKER)
    mesh = plsc.VectorSubcoreMesh(core_axis_name="c", subcore_axis_name="s")
    run = pl.kernel(
        _sc_body,
        out_type=jax.ShapeDtypeStruct((DIM, BATCH), jnp.float32),
        mesh=mesh,
        scratch_types=[
            pltpu.VMEM((1, COLS_PER_WORKER + LANES), jnp.int32),
            pltpu.VMEM((DIM, COLS_PER_WORKER), jnp.float32),
            pltpu.VMEM_SHARED(
                (NUM_SUBCORES, RING, DIM, LANE_TILE), jnp.float32
            ),
            pltpu.VMEM((DIM,), jnp.float32),
            pltpu.SemaphoreType.DMA((RING,)),
            pltpu.SemaphoreType.DMA,
        ],
        compiler_params=pltpu.CompilerParams(needs_layout_passes=False),
    )
    out_t = run(x.T, cls_r, biases.T)
    return out_t.T


# speedup vs baseline: 1.5870x; 1.5870x over previous
import jax
import jax.numpy as jnp
from jax import lax
from jax.experimental import pallas as pl
from jax.experimental.pallas import tpu as pltpu
from jax.experimental.pallas import tpu_sc as plsc

BATCH = 16384
DIM = 64
N_CLASSES = 1000000
NUM_CORES = 2
NUM_SUBCORES = 16
NUM_WORKERS = NUM_CORES * NUM_SUBCORES
COLS_PER_WORKER = BATCH // NUM_WORKERS
LANES = 16
LANE_TILE = 128
RING = 11


def _sc_body(xt_hbm, cls_hbm, pt_hbm, out_hbm, cls_v, acc, bufs, sems):
    wid = lax.axis_index("s") * NUM_CORES + lax.axis_index("c")
    base = pl.multiple_of(wid * COLS_PER_WORKER, COLS_PER_WORKER)
    pltpu.sync_copy(cls_hbm.at[wid], cls_v.at[:, pl.ds(0, COLS_PER_WORKER)])
    pltpu.sync_copy(xt_hbm.at[:, pl.ds(base, COLS_PER_WORKER)], acc)

    def fire(c, k):
        z = cls_v[0, pl.ds(c, LANES)][0]
        z128 = pl.multiple_of(
            lax.shift_left(lax.shift_right_logical(z, 7), 7), LANE_TILE
        )
        pltpu.async_copy(
            pt_hbm.at[:, pl.ds(z128, LANE_TILE)], bufs.at[k], sems.at[k]
        )

    def drain(k):
        pltpu.make_async_copy(
            pt_hbm.at[:, pl.ds(0, LANE_TILE)], bufs.at[k], sems.at[k]
        ).wait()

    for k in range(RING):
        fire(k, k)

    def col_body(c, carry):
        k = lax.rem(c, RING)
        drain(k)
        l = cls_v[0, pl.ds(c, LANES)][0] & (LANE_TILE - 1)
        lvec = jnp.full((LANES,), l, jnp.int32)
        kvec = jnp.full((LANES,), k, jnp.int32)
        cvec = jnp.full((LANES,), c, jnp.int32)
        for j in range(DIM // LANES):
            dvec = lax.iota(jnp.int32, LANES) + j * LANES
            v = plsc.load_gather(bufs, [kvec, dvec, lvec])
            plsc.addupdate_scatter(acc, [dvec, cvec], v)

        @pl.when(c + RING < COLS_PER_WORKER)
        def _():
            fire(c + RING, k)

        return carry

    lax.fori_loop(0, COLS_PER_WORKER, col_body, 0)
    pltpu.sync_copy(acc, out_hbm.at[:, pl.ds(base, COLS_PER_WORKER)])


@jax.jit
def kernel(x, classes, biases):
    cls_r = classes.astype(jnp.int32).reshape(NUM_WORKERS, 1, COLS_PER_WORKER)
    mesh = plsc.VectorSubcoreMesh(core_axis_name="c", subcore_axis_name="s")
    run = pl.kernel(
        _sc_body,
        out_type=jax.ShapeDtypeStruct((DIM, BATCH), jnp.float32),
        mesh=mesh,
        scratch_types=[
            pltpu.VMEM((1, COLS_PER_WORKER + LANES), jnp.int32),
            pltpu.VMEM((DIM, COLS_PER_WORKER), jnp.float32),
            pltpu.VMEM((RING, DIM, LANE_TILE), jnp.float32),
            pltpu.SemaphoreType.DMA((RING,)),
        ],
        compiler_params=pltpu.CompilerParams(needs_layout_passes=False),
    )
    out_t = run(x.T, cls_r, biases.T)
    return out_t.T


# speedup vs baseline: 2.2514x; 1.4186x over previous
import jax
import jax.numpy as jnp
from jax import lax
from jax.experimental import pallas as pl
from jax.experimental.pallas import tpu as pltpu
from jax.experimental.pallas import tpu_sc as plsc

BATCH = 16384
DIM = 64
N_CLASSES = 1000000
NUM_CORES = 2
NUM_SUBCORES = 16
NUM_WORKERS = NUM_CORES * NUM_SUBCORES
COLS_PER_WORKER = BATCH // NUM_WORKERS
LANES = 16
LANE_TILE = 128
RING = 10
FLUSH = 128
STAGE_ROWS = BATCH + FLUSH
NLOC = COLS_PER_WORKER
NGRP_LOC = NLOC // LANES
ZOFF = LANE_TILE


def _iota16():
    return lax.iota(jnp.int32, LANES)


def _a_body(zs_hbm, ns_hbm, pt_hbm, stage_hbm,
            zs, ns, flist, fstart, bufs, outbuf, nlist, sems, fsem):
    w = lax.axis_index("s") * NUM_CORES + lax.axis_index("c")
    pltpu.sync_copy(zs_hbm.at[w], zs.at[:, pl.ds(ZOFF, NLOC)])
    pltpu.sync_copy(ns_hbm.at[w], ns.at[:, pl.ds(0, NLOC)])
    plsc.store_scatter(
        zs.at[0],
        [_iota16() + (ZOFF - LANES)],
        jnp.full((LANES,), -(1 << 30), jnp.int32),
        mask=_iota16() == LANES - 1,
    )

    def dedup_body(g, nf):
        cur = zs[0, pl.ds(ZOFF + g * LANES, LANES)]
        prev = zs[0, pl.ds(ZOFF - 1 + g * LANES, LANES)]
        tcur = lax.shift_right_logical(cur, 7)
        tprev = lax.shift_right_logical(prev, 7)
        mask = tcur != tprev
        mi = plsc.cumsum(mask.astype(jnp.int32))
        slots = mi + (nf - 1)
        plsc.store_scatter(flist, [slots], lax.shift_left(tcur, 7), mask=mask)
        plsc.store_scatter(fstart, [slots], _iota16() + g * LANES, mask=mask)
        return nf + mi[LANES - 1]

    nf = lax.fori_loop(0, NGRP_LOC, dedup_body, 0)
    plsc.store_scatter(
        fstart,
        [jnp.full((LANES,), nf, jnp.int32)],
        jnp.full((LANES,), NLOC, jnp.int32),
        mask=_iota16() == 0,
    )

    def preset_nlist():
        for k in range(FLUSH // LANES):
            nlist[pl.ds(k * LANES, LANES)] = _iota16() + (BATCH + k * LANES)

    preset_nlist()

    def flush():
        pltpu.async_copy(outbuf, stage_hbm.at[nlist], fsem).wait()
        preset_nlist()

    def fire(f, k):
        z128 = pl.multiple_of(flist[pl.ds(f, LANES)][0], LANE_TILE)
        pltpu.async_copy(
            pt_hbm.at[:, pl.ds(z128, LANE_TILE)], bufs.at[k], sems.at[k]
        )

    def drain(k):
        pltpu.make_async_copy(
            pt_hbm.at[:, pl.ds(0, LANE_TILE)], bufs.at[k], sems.at[k]
        ).wait()

    for k in range(RING):
        @pl.when(k < nf)
        def _():
            fire(k, k)

    def fetch_body(f, s):
        k = lax.rem(f, RING)
        drain(k)

        @pl.when(f + RING < nf)
        def _():
            fire(f + RING, k)

        kvec = jnp.full((LANES,), k, jnp.int32)
        t0 = fstart[pl.ds(f, LANES)][0]
        t1 = fstart[pl.ds(f + 1, LANES)][0]

        def cls_body(t, s2):
            l = zs[0, pl.ds(ZOFF + t, LANES)][0] & (LANE_TILE - 1)
            n = ns[0, pl.ds(t, LANES)][0]
            lvec = jnp.full((LANES,), l, jnp.int32)
            for j in range(DIM // LANES):
                dvec = _iota16() + j * LANES
                v = plsc.load_gather(bufs, [kvec, dvec, lvec])
                outbuf[s2, pl.ds(j * LANES, LANES)] = v
            plsc.store_scatter(
                nlist,
                [jnp.full((LANES,), s2, jnp.int32)],
                jnp.full((LANES,), n, jnp.int32),
                mask=_iota16() == 0,
            )
            s3 = s2 + 1

            @pl.when(s3 == FLUSH)
            def _():
                flush()

            return jnp.where(s3 == FLUSH, 0, s3)

        return lax.fori_loop(t0, t1, cls_body, s)

    s_end = lax.fori_loop(0, nf, fetch_body, 0)

    @pl.when(s_end > 0)
    def _():
        flush()


def _b_body(xt_hbm, stage_hbm, out_hbm, acc, piece):
    w = lax.axis_index("s") * NUM_CORES + lax.axis_index("c")
    base = pl.multiple_of(w * COLS_PER_WORKER, COLS_PER_WORKER)
    pltpu.sync_copy(xt_hbm.at[:, pl.ds(base, COLS_PER_WORKER)], acc)
    half = COLS_PER_WORKER // 2
    for p in range(2):
        pltpu.sync_copy(
            stage_hbm.at[pl.ds(base + p * half, half), :], piece
        )

        def row_body(r, carry):
            rvec = jnp.full((LANES,), r, jnp.int32)
            cvec = jnp.full((LANES,), p * half + r, jnp.int32)
            for j in range(DIM // LANES):
                dvec = _iota16() + j * LANES
                v = plsc.load_gather(piece, [rvec, dvec])
                plsc.addupdate_scatter(acc, [dvec, cvec], v)
            return carry

        lax.fori_loop(0, half, row_body, 0)
    pltpu.sync_copy(acc, out_hbm.at[:, pl.ds(base, COLS_PER_WORKER)])


@jax.jit
def kernel(x, classes, biases):
    cls32 = classes.astype(jnp.int32)
    nidx = lax.iota(jnp.int32, BATCH)
    zs_sorted, ns_sorted = lax.sort_key_val(cls32, nidx)
    zs_r = zs_sorted.reshape(NUM_WORKERS, 1, NLOC)
    ns_r = ns_sorted.reshape(NUM_WORKERS, 1, NLOC)

    mesh = plsc.VectorSubcoreMesh(core_axis_name="c", subcore_axis_name="s")
    run_a = pl.kernel(
        _a_body,
        out_type=jax.ShapeDtypeStruct((STAGE_ROWS, LANE_TILE), jnp.float32),
        mesh=mesh,
        scratch_types=[
            pltpu.VMEM((1, ZOFF + NLOC + LANES), jnp.int32),
            pltpu.VMEM((1, NLOC + LANES), jnp.int32),
            pltpu.VMEM((NLOC + LANES,), jnp.int32),
            pltpu.VMEM((NLOC + 2 * LANES,), jnp.int32),
            pltpu.VMEM((RING, DIM, LANE_TILE), jnp.float32),
            pltpu.VMEM((FLUSH, LANE_TILE), jnp.float32),
            pltpu.VMEM((FLUSH,), jnp.int32),
            pltpu.SemaphoreType.DMA((RING,)),
            pltpu.SemaphoreType.DMA,
        ],
        compiler_params=pltpu.CompilerParams(needs_layout_passes=False),
    )
    staging = run_a(zs_r, ns_r, biases.T)

    run_b = pl.kernel(
        _b_body,
        out_type=jax.ShapeDtypeStruct((DIM, BATCH), jnp.float32),
        mesh=mesh,
        scratch_types=[
            pltpu.VMEM((DIM, COLS_PER_WORKER), jnp.float32),
            pltpu.VMEM((COLS_PER_WORKER // 2, LANE_TILE), jnp.float32),
        ],
        compiler_params=pltpu.CompilerParams(needs_layout_passes=False),
    )
    out_t = run_b(x.T, staging)
    return out_t.T
